# baseline (speedup 1.0000x reference)
import functools

import jax
import jax.numpy as jnp
from jax import lax
from jax.experimental import pallas as pl
from jax.experimental.pallas import tpu as pltpu
from jax.experimental.pallas import tpu_sc as plsc

_B = 4096
_H = 200
_D = 32
_N = _B * _H
_NC, _NS = 2, 16
_NW = _NC * _NS
_G = 128
_ROWS = _N // _G
_ROWS_PER_W = _ROWS // _NW
_RPC = 10
_CHUNK = _RPC * _G
_NCHUNK = _ROWS_PER_W // _RPC


def _sc_gather(idx_flat, table):
    mesh = plsc.VectorSubcoreMesh(core_axis_name="c", subcore_axis_name="s")

    @functools.partial(
        pl.kernel,
        mesh=mesh,
        compiler_params=pltpu.CompilerParams(use_tc_tiling_on_sc=False),
        out_type=jax.ShapeDtypeStruct((_N, _D), jnp.float32),
        scratch_types=[
            pltpu.VMEM((_ROWS_PER_W * _G,), jnp.int32),
            pltpu.VMEM((2 * _CHUNK, _D), jnp.float32),
            pltpu.SemaphoreType.DMA,
            pltpu.SemaphoreType.DMA,
            pltpu.SemaphoreType.DMA,
            pltpu.SemaphoreType.DMA,
        ],
    )
    def k(idx_hbm, table_hbm, out_hbm, idx_v, rows_v,
          sem_g0, sem_g1, sem_s0, sem_s1):
        wid = lax.axis_index("s") * _NC + lax.axis_index("c")
        row0 = wid * _ROWS_PER_W
        base = row0 * _G

        pltpu.sync_copy(idx_hbm.at[pl.ds(base, _ROWS_PER_W * _G)], idx_v)

        def start_gather(c, off, sem):
            pltpu.async_copy(
                table_hbm.at[idx_v.at[pl.ds(c * _CHUNK, _CHUNK)]],
                rows_v.at[pl.ds(off, _CHUNK)],
                sem,
            )

        def wait_store(c, off, sem):
            pltpu.make_async_copy(
                rows_v.at[pl.ds(off, _CHUNK)],
                out_hbm.at[pl.ds(base + c * _CHUNK, _CHUNK)],
                sem,
            ).wait()

        start_gather(0, 0, sem_g0)

        def body(g, carry):
            nxt = g + 1
            even_n = lax.rem(nxt, 2) == 0
            even_g = lax.rem(g, 2) == 0

            @pl.when(jnp.logical_and(nxt < _NCHUNK, even_n))
            def _():
                wait_store(g - 1, 0, sem_s0)
                start_gather(nxt, 0, sem_g0)

            @pl.when(jnp.logical_and(nxt < _NCHUNK, jnp.logical_not(even_n)))
            def _():
                @pl.when(g >= 1)
                def _():
                    wait_store(g - 1, _CHUNK, sem_s1)

                start_gather(nxt, _CHUNK, sem_g1)

            @pl.when(even_g)
            def _():
                pltpu.make_async_copy(
                    out_hbm.at[pl.ds(0, _CHUNK)],
                    rows_v.at[pl.ds(0, _CHUNK)],
                    sem_g0,
                ).wait()
                pltpu.async_copy(
                    rows_v.at[pl.ds(0, _CHUNK)],
                    out_hbm.at[pl.ds(base + g * _CHUNK, _CHUNK)],
                    sem_s0,
                )

            @pl.when(jnp.logical_not(even_g))
            def _():
                pltpu.make_async_copy(
                    out_hbm.at[pl.ds(0, _CHUNK)],
                    rows_v.at[pl.ds(_CHUNK, _CHUNK)],
                    sem_g1,
                ).wait()
                pltpu.async_copy(
                    rows_v.at[pl.ds(_CHUNK, _CHUNK)],
                    out_hbm.at[pl.ds(base + g * _CHUNK, _CHUNK)],
                    sem_s1,
                )

            return carry

        lax.fori_loop(0, _NCHUNK, body, 0)

        pltpu.make_async_copy(
            rows_v.at[pl.ds(0, _CHUNK)],
            out_hbm.at[pl.ds(base + (_NCHUNK - 2) * _CHUNK, _CHUNK)],
            sem_s0,
        ).wait()
        pltpu.make_async_copy(
            rows_v.at[pl.ds(_CHUNK, _CHUNK)],
            out_hbm.at[pl.ds(base + (_NCHUNK - 1) * _CHUNK, _CHUNK)],
            sem_s1,
        ).wait()

    return k(idx_flat, table)


_V = 1000000
_FULL = 7812
_TAIL_C = 7812
_TAIL_W = _V - _FULL * 128
_CB = 4
_KMAIN = 61
_NBATCH = _NW * _KMAIN
_BLK = 128 * _D
_BBLK = _CB * _BLK
_PADW = _CB * 128 + 5


def _sc_transpose(table_t, tail_flat):
    mesh = plsc.VectorSubcoreMesh(core_axis_name="c", subcore_axis_name="s")

    @functools.partial(
        pl.kernel,
        mesh=mesh,
        compiler_params=pltpu.CompilerParams(
            use_tc_tiling_on_sc=True, needs_layout_passes=False),
        out_type=jax.ShapeDtypeStruct((_V * _D,), jnp.float32),
        scratch_types=[
            pltpu.VMEM((32, _PADW), jnp.float32),
            pltpu.VMEM((32, _PADW), jnp.float32),
            pltpu.VMEM((_BBLK,), jnp.float32),
            pltpu.VMEM((_BBLK,), jnp.float32),
            pltpu.SemaphoreType.DMA,
            pltpu.SemaphoreType.DMA,
            pltpu.SemaphoreType.DMA,
            pltpu.SemaphoreType.DMA,
        ],
    )
    def k(tt_hbm, tail_hbm, out_hbm,
          ina, inb, ob0, ob1,
          sem_ia, sem_ib, sem_oa, sem_ob):
        wid = lax.axis_index("s") * _NC + lax.axis_index("c")
        iota16 = lax.iota(jnp.int32, 16)
        rows_lo = iota16
        rows_hi = iota16 + 16

        slot_a = (ina, ob0, sem_ia, sem_oa)
        slot_b = (inb, ob1, sem_ib, sem_ob)

        def fetch(col0, buf, sem, width=_CB * 128):
            for dt in range(4):
                pltpu.async_copy(
                    tt_hbm.at[pl.ds(dt * 8, 8), pl.ds(col0, width)],
                    buf.at[pl.ds(dt * 8, 8), pl.ds(0, width)],
                    sem,
                )

        def drain_fetch(buf, sem, width=_CB * 128):
            for dt in range(4):
                pltpu.make_async_copy(
                    tt_hbm.at[pl.ds(0, 8), pl.ds(0, width)],
                    buf.at[pl.ds(dt * 8, 8), pl.ds(0, width)],
                    sem,
                ).wait()

        def transpose_cols(buf, ob, j):
            for v in range(128):
                col = jnp.full((16,), j * 128 + v, jnp.int32)
                x0 = plsc.load_gather(buf, [rows_lo, col])
                x1 = plsc.load_gather(buf, [rows_hi, col])
                dst = j * _BLK + v * _D
                ob[pl.ds(dst, 16)] = x0
                ob[pl.ds(dst + 16, 16)] = x1

        def transpose_batch(buf, ob):
            def jbody(j, carry):
                transpose_cols(buf, ob, j)
                return carry

            lax.fori_loop(0, _CB, jbody, 0)

        def process(m, slot, have_next, next_m, other, k_ge2, prev_m):
            buf, ob, sem_i, sem_o = slot

            @pl.when(have_next)
            def _():
                fetch(next_m * _CB * 128, other[0], other[2])

            drain_fetch(buf, sem_i)

            @pl.when(k_ge2)
            def _():
                pltpu.make_async_copy(
                    ob, out_hbm.at[pl.ds(prev_m * _BBLK, _BBLK)], sem_o,
                ).wait()

            transpose_batch(buf, ob)
            pltpu.async_copy(ob, out_hbm.at[pl.ds(m * _BBLK, _BBLK)], sem_o)

        fetch(wid * _CB * 128, ina, sem_ia)

        def body(kk, carry):
            m = wid + kk * _NW
            have_next = kk + 1 < _KMAIN
            even = lax.rem(kk, 2) == 0

            @pl.when(even)
            def _():
                process(m, slot_a, have_next, m + _NW, slot_b,
                        kk >= 2, m - 2 * _NW)

            @pl.when(jnp.logical_not(even))
            def _():
                process(m, slot_b, have_next, m + _NW, slot_a,
                        kk >= 2, m - 2 * _NW)

            return carry

        lax.fori_loop(0, _KMAIN, body, 0)

        pltpu.make_async_copy(
            ob1, out_hbm.at[pl.ds((wid + (_KMAIN - 2) * _NW) * _BBLK, _BBLK)],
            sem_ob,
        ).wait()
        pltpu.make_async_copy(
            ob0, out_hbm.at[pl.ds((wid + (_KMAIN - 1) * _NW) * _BBLK, _BBLK)],
            sem_oa,
        ).wait()

        @pl.when(wid < 4)
        def _():
            c = _NBATCH * _CB + wid
            fetch(c * 128, ina, sem_ia, width=128)
            drain_fetch(ina, sem_ia, width=128)
            transpose_cols(ina, ob0, 0)
            pltpu.async_copy(
                ob0.at[pl.ds(0, _BLK)],
                out_hbm.at[pl.ds(c * _BLK, _BLK)], sem_oa)
            pltpu.make_async_copy(
                ob0.at[pl.ds(0, _BLK)],
                out_hbm.at[pl.ds(c * _BLK, _BLK)], sem_oa).wait()

        @pl.when(wid == 4)
        def _():
            nel = _TAIL_W * _D
            pltpu.sync_copy(tail_hbm, ob1.at[pl.ds(0, nel)])
            pltpu.sync_copy(ob1.at[pl.ds(0, nel)],
                            out_hbm.at[pl.ds(_TAIL_C * _BLK, nel)])

    return k(table_t, tail_flat)


def _tc_transpose_wide(table_t):
    blk = 4096

    def body(tin, tout):
        x = tin[...].reshape(_D, blk // 4, 4)
        tout[...] = jnp.transpose(x, (1, 2, 0)).reshape(blk // 4, 4 * _D)

    return pl.pallas_call(
        body,
        grid=(_V // blk + 1,),
        in_specs=[pl.BlockSpec((_D, blk), lambda i: (0, i))],
        out_specs=pl.BlockSpec((blk // 4, 4 * _D), lambda i: (i, 0)),
        out_shape=jax.ShapeDtypeStruct((_V // 4, 4 * _D), jnp.float32),
    )(table_t)


def kernel(input_seqs, table):
    idx_flat = input_seqs.reshape(_N).astype(jnp.int32)
    table_rm = _tc_transpose_wide(table.T).reshape(_V, _D)
    out = _sc_gather(idx_flat, table_rm)
    return out.reshape(_B, _H, _D)


# speedup vs baseline: 2.9876x; 2.9876x over previous
import functools

import jax
import jax.numpy as jnp
from jax import lax
from jax.experimental import pallas as pl
from jax.experimental.pallas import tpu as pltpu
from jax.experimental.pallas import tpu_sc as plsc

_B = 4096
_H = 200
_D = 32
_N = _B * _H
_NC, _NS = 2, 16
_NW = _NC * _NS
_N_PER_W = _N // _NW
_CHUNK = 1280
_NCHUNK = _N_PER_W // _CHUNK


def _sc_gather(idx_flat, table):
    mesh = plsc.VectorSubcoreMesh(core_axis_name="c", subcore_axis_name="s")

    @functools.partial(
        pl.kernel,
        mesh=mesh,
        compiler_params=pltpu.CompilerParams(use_tc_tiling_on_sc=False),
        out_type=jax.ShapeDtypeStruct((_N, _D), jnp.float32),
        scratch_types=[
            pltpu.VMEM((_N_PER_W,), jnp.int32),
            pltpu.VMEM((2 * _CHUNK, _D), jnp.float32),
            pltpu.SemaphoreType.DMA,
            pltpu.SemaphoreType.DMA,
            pltpu.SemaphoreType.DMA,
            pltpu.SemaphoreType.DMA,
        ],
    )
    def k(idx_hbm, table_hbm, out_hbm, idx_v, rows_v,
          sem_g0, sem_g1, sem_s0, sem_s1):
        wid = lax.axis_index("s") * _NC + lax.axis_index("c")
        base = wid * _N_PER_W

        pltpu.sync_copy(idx_hbm.at[pl.ds(base, _N_PER_W)], idx_v)

        def start_gather(c, off, sem):
            pltpu.async_copy(
                table_hbm.at[idx_v.at[pl.ds(c * _CHUNK, _CHUNK)]],
                rows_v.at[pl.ds(off, _CHUNK)],
                sem,
            )

        def wait_store(c, off, sem):
            pltpu.make_async_copy(
                rows_v.at[pl.ds(off, _CHUNK)],
                out_hbm.at[pl.ds(base + c * _CHUNK, _CHUNK)],
                sem,
            ).wait()

        start_gather(0, 0, sem_g0)

        def body(g, carry):
            nxt = g + 1
            even_n = lax.rem(nxt, 2) == 0
            even_g = lax.rem(g, 2) == 0

            @pl.when(jnp.logical_and(nxt < _NCHUNK, even_n))
            def _():
                wait_store(g - 1, 0, sem_s0)
                start_gather(nxt, 0, sem_g0)

            @pl.when(jnp.logical_and(nxt < _NCHUNK, jnp.logical_not(even_n)))
            def _():
                @pl.when(g >= 1)
                def _():
                    wait_store(g - 1, _CHUNK, sem_s1)

                start_gather(nxt, _CHUNK, sem_g1)

            @pl.when(even_g)
            def _():
                pltpu.make_async_copy(
                    out_hbm.at[pl.ds(0, _CHUNK)],
                    rows_v.at[pl.ds(0, _CHUNK)],
                    sem_g0,
                ).wait()
                pltpu.async_copy(
                    rows_v.at[pl.ds(0, _CHUNK)],
                    out_hbm.at[pl.ds(base + g * _CHUNK, _CHUNK)],
                    sem_s0,
                )

            @pl.when(jnp.logical_not(even_g))
            def _():
                pltpu.make_async_copy(
                    out_hbm.at[pl.ds(0, _CHUNK)],
                    rows_v.at[pl.ds(_CHUNK, _CHUNK)],
                    sem_g1,
                ).wait()
                pltpu.async_copy(
                    rows_v.at[pl.ds(_CHUNK, _CHUNK)],
                    out_hbm.at[pl.ds(base + g * _CHUNK, _CHUNK)],
                    sem_s1,
                )

            return carry

        lax.fori_loop(0, _NCHUNK, body, 0)

        pltpu.make_async_copy(
            rows_v.at[pl.ds(0, _CHUNK)],
            out_hbm.at[pl.ds(base + (_NCHUNK - 2) * _CHUNK, _CHUNK)],
            sem_s0,
        ).wait()
        pltpu.make_async_copy(
            rows_v.at[pl.ds(_CHUNK, _CHUNK)],
            out_hbm.at[pl.ds(base + (_NCHUNK - 1) * _CHUNK, _CHUNK)],
            sem_s1,
        ).wait()

    return k(idx_flat, table)


def kernel(input_seqs, table):
    idx_flat = input_seqs.reshape(_N).astype(jnp.int32)
    out = _sc_gather(idx_flat, table)
    return out.reshape(_B, _H, _D)
